# Initial kernel scaffold; baseline (speedup 1.0000x reference)
import jax
import jax.numpy as jnp
from jax.experimental import pallas as pl


def kernel(x, edge_index, W, att_src, att_dst, bias):
    raise NotImplementedError("write your pallas kernel here")


# speedup vs baseline: 12.9542x; 12.9542x over previous
import jax
import jax.numpy as jnp
from jax.experimental import pallas as pl

_NEG_SLOPE = 0.2


def _count_body(src_ref, dst_ref, o_ref):
    i = pl.program_id(0)
    ch = src_ref.shape[2]
    n = o_ref.shape[0]
    s = src_ref[0, 0, :]
    d = dst_ref[0, 0, :]
    node_rows = jax.lax.broadcasted_iota(jnp.int32, (n, ch), 0)
    node_cols = jax.lax.broadcasted_iota(jnp.int32, (ch, n), 1)
    d_oh = (node_rows == d[None, :]).astype(jnp.bfloat16)
    s_oh = (s[:, None] == node_cols).astype(jnp.bfloat16)
    contrib = jnp.dot(d_oh, s_oh, preferred_element_type=jnp.float32)

    @pl.when(i == 0)
    def _():
        r = jax.lax.broadcasted_iota(jnp.int32, (n, n), 0)
        c = jax.lax.broadcasted_iota(jnp.int32, (n, n), 1)
        eye = (r == c).astype(jnp.float32)
        o_ref[...] = eye + contrib

    @pl.when(i != 0)
    def _():
        o_ref[...] += contrib


def _make_gat_body(heads):
    def _gat_body(x_ref, cnt_ref, w_ref, as_ref, ad_ref, b_ref, o_ref):
        f = w_ref.shape[1]
        o = f // heads
        xb = x_ref[0]
        h = jnp.dot(xb.astype(jnp.bfloat16), w_ref[...].astype(jnp.bfloat16),
                    preferred_element_type=jnp.float32)
        ps = h * as_ref[0][None, :]
        pd = h * ad_ref[0][None, :]
        cnt = cnt_ref[...]
        mask = cnt > 0.0
        hb = h.astype(jnp.bfloat16)
        for hd in range(heads):
            sl = slice(hd * o, (hd + 1) * o)
            a_s = jnp.sum(ps[:, sl], axis=1)
            a_d = jnp.sum(pd[:, sl], axis=1)
            e = a_d[:, None] + a_s[None, :]
            e = jnp.where(e > 0, e, _NEG_SLOPE * e)
            em = jnp.max(jnp.where(mask, e, -1e30), axis=1, keepdims=True)
            ex = jnp.where(mask, cnt * jnp.exp(e - em), 0.0)
            den = jnp.sum(ex, axis=1, keepdims=True)
            att = (ex / (den + 1e-16)).astype(jnp.bfloat16)
            oh = jnp.dot(att, hb[:, sl], preferred_element_type=jnp.float32)
            ob = oh + b_ref[0, sl][None, :]
            o_ref[0, :, sl] = jnp.where(ob > 0, ob, _NEG_SLOPE * ob)

    return _gat_body


def kernel(x, edge_index, W, att_src, att_dst, bias):
    B, C, T, D = x.shape
    F = W.shape[1]
    H = att_src.shape[0]
    N = C
    E = edge_index.shape[1]
    ei = edge_index.astype(jnp.int32)

    ch = min(512, E)
    pad = (-E) % ch
    if pad:
        ei = jnp.concatenate(
            [ei, jnp.full((2, pad), -1, jnp.int32)], axis=1)
    nch = (E + pad) // ch
    srcs = ei[0].reshape(nch, 1, ch)
    dsts = ei[1].reshape(nch, 1, ch)

    cnt = pl.pallas_call(
        _count_body,
        grid=(nch,),
        in_specs=[
            pl.BlockSpec((1, 1, ch), lambda i: (i, 0, 0)),
            pl.BlockSpec((1, 1, ch), lambda i: (i, 0, 0)),
        ],
        out_specs=pl.BlockSpec((N, N), lambda i: (0, 0)),
        out_shape=jax.ShapeDtypeStruct((N, N), jnp.float32),
    )(srcs, dsts)

    asf = att_src.reshape(1, F)
    adf = att_dst.reshape(1, F)
    bf = bias.reshape(1, F)

    xs = jnp.transpose(x, (0, 2, 1, 3)).reshape(B * T, C, D)
    out = pl.pallas_call(
        _make_gat_body(H),
        grid=(B * T,),
        in_specs=[
            pl.BlockSpec((1, C, D), lambda i: (i, 0, 0)),
            pl.BlockSpec((N, N), lambda i: (0, 0)),
            pl.BlockSpec((D, F), lambda i: (0, 0)),
            pl.BlockSpec((1, F), lambda i: (0, 0)),
            pl.BlockSpec((1, F), lambda i: (0, 0)),
            pl.BlockSpec((1, F), lambda i: (0, 0)),
        ],
        out_specs=pl.BlockSpec((1, C, F), lambda i: (i, 0, 0)),
        out_shape=jax.ShapeDtypeStruct((B * T, C, F), jnp.float32),
    )(xs, cnt, W, asf, adf, bf)
    return out.reshape(B, T, C, F).transpose(0, 2, 1, 3)


# speedup vs baseline: 18.6493x; 1.4396x over previous
import jax
import jax.numpy as jnp
from jax.experimental import pallas as pl

_NEG_SLOPE = 0.2
_NEG_BIG = -1e30


def _count_body(src_ref, dst_ref, cnt_ref, log_ref):
    i = pl.program_id(0)
    nsteps = pl.num_programs(0)
    ch = src_ref.shape[2]
    n = cnt_ref.shape[0]
    s = src_ref[0, 0, :]
    d = dst_ref[0, 0, :]
    node_rows = jax.lax.broadcasted_iota(jnp.int32, (n, ch), 0)
    node_cols = jax.lax.broadcasted_iota(jnp.int32, (ch, n), 1)
    d_oh = (node_rows == d[None, :]).astype(jnp.bfloat16)
    s_oh = (s[:, None] == node_cols).astype(jnp.bfloat16)
    contrib = jnp.dot(d_oh, s_oh, preferred_element_type=jnp.float32)

    @pl.when(i == 0)
    def _():
        r = jax.lax.broadcasted_iota(jnp.int32, (n, n), 0)
        c = jax.lax.broadcasted_iota(jnp.int32, (n, n), 1)
        eye = (r == c).astype(jnp.float32)
        cnt_ref[...] = eye + contrib

    @pl.when(i != 0)
    def _():
        cnt_ref[...] += contrib

    @pl.when(i == nsteps - 1)
    def _():
        c = cnt_ref[...]
        log_ref[...] = jnp.where(c > 0.0, jnp.log(c), _NEG_BIG)


def _make_gat_body(heads):
    def _gat_body(x_ref, log_ref, w_ref, amat_ref, b_ref, o_ref):
        f = w_ref.shape[1]
        o = f // heads
        xb = x_ref[0]
        h = jnp.dot(xb.astype(jnp.bfloat16), w_ref[...].astype(jnp.bfloat16),
                    preferred_element_type=jnp.float32)
        hb = h.astype(jnp.bfloat16)
        al = jnp.dot(hb, amat_ref[...], preferred_element_type=jnp.float32)
        alt = al.T
        lcnt = log_ref[...]
        for hd in range(heads):
            a_s = alt[hd:hd + 1, :]
            a_d = al[:, heads + hd:heads + hd + 1]
            t = a_d + a_s
            el = jnp.maximum(t, _NEG_SLOPE * t) + lcnt
            em = jnp.max(el, axis=1, keepdims=True)
            ex = jnp.exp(el - em)
            den = jnp.sum(ex, axis=1, keepdims=True)
            oh = jnp.dot(ex.astype(jnp.bfloat16), hb[:, hd * o:(hd + 1) * o],
                         preferred_element_type=jnp.float32)
            ob = oh * (1.0 / (den + 1e-16)) + b_ref[0, hd * o:(hd + 1) * o][None, :]
            o_ref[0, :, hd * o:(hd + 1) * o] = jnp.maximum(ob, _NEG_SLOPE * ob)

    return _gat_body


def kernel(x, edge_index, W, att_src, att_dst, bias):
    B, C, T, D = x.shape
    F = W.shape[1]
    H = att_src.shape[0]
    O = F // H
    N = C
    E = edge_index.shape[1]
    ei = edge_index.astype(jnp.int32)

    ch = min(512, E)
    pad = (-E) % ch
    if pad:
        ei = jnp.concatenate(
            [ei, jnp.full((2, pad), -1, jnp.int32)], axis=1)
    nch = (E + pad) // ch
    srcs = ei[0].reshape(nch, 1, ch)
    dsts = ei[1].reshape(nch, 1, ch)

    _, lcnt = pl.pallas_call(
        _count_body,
        grid=(nch,),
        in_specs=[
            pl.BlockSpec((1, 1, ch), lambda i: (i, 0, 0)),
            pl.BlockSpec((1, 1, ch), lambda i: (i, 0, 0)),
        ],
        out_specs=[
            pl.BlockSpec((N, N), lambda i: (0, 0)),
            pl.BlockSpec((N, N), lambda i: (0, 0)),
        ],
        out_shape=[
            jax.ShapeDtypeStruct((N, N), jnp.float32),
            jax.ShapeDtypeStruct((N, N), jnp.float32),
        ],
    )(srcs, dsts)

    asf = att_src.reshape(F)
    adf = att_dst.reshape(F)
    fhead = jnp.arange(F, dtype=jnp.int32) // O
    sel = (fhead[:, None] == jnp.arange(H, dtype=jnp.int32)[None, :])
    amat = jnp.concatenate(
        [jnp.where(sel, asf[:, None], 0.0), jnp.where(sel, adf[:, None], 0.0)],
        axis=1).astype(jnp.bfloat16)
    bf = bias.reshape(1, F)

    xv = x.reshape(B, C, T * D)
    out = pl.pallas_call(
        _make_gat_body(H),
        grid=(B * T,),
        in_specs=[
            pl.BlockSpec((1, C, D), lambda i: (i // T, 0, i % T)),
            pl.BlockSpec((N, N), lambda i: (0, 0)),
            pl.BlockSpec((D, F), lambda i: (0, 0)),
            pl.BlockSpec((F, 2 * H), lambda i: (0, 0)),
            pl.BlockSpec((1, F), lambda i: (0, 0)),
        ],
        out_specs=pl.BlockSpec((1, C, F), lambda i: (i // T, 0, i % T)),
        out_shape=jax.ShapeDtypeStruct((B, C, T * F), jnp.float32),
    )(xv, lcnt, W, amat, bf)
    return out.reshape(B, C, T, F)


# speedup vs baseline: 21.1467x; 1.1339x over previous
import jax
import jax.numpy as jnp
from jax.experimental import pallas as pl
from jax.experimental.pallas import tpu as pltpu

_NEG_SLOPE = 0.2


def _count_body(src_ref, dst_ref, cnt_ref):
    i = pl.program_id(0)
    ch = src_ref.shape[2]
    n = cnt_ref.shape[0]
    s = src_ref[0, 0, :]
    d = dst_ref[0, 0, :]
    node_rows = jax.lax.broadcasted_iota(jnp.int32, (n, ch), 0)
    node_cols = jax.lax.broadcasted_iota(jnp.int32, (ch, n), 1)
    d_oh = (node_rows == d[None, :]).astype(jnp.bfloat16)
    s_oh = (s[:, None] == node_cols).astype(jnp.bfloat16)
    contrib = jnp.dot(d_oh, s_oh, preferred_element_type=jnp.float32)

    @pl.when(i == 0)
    def _():
        r = jax.lax.broadcasted_iota(jnp.int32, (n, n), 0)
        c = jax.lax.broadcasted_iota(jnp.int32, (n, n), 1)
        eye = (r == c).astype(jnp.float32)
        cnt_ref[...] = eye + contrib

    @pl.when(i != 0)
    def _():
        cnt_ref[...] += contrib


def _make_gat_body(heads):
    def _gat_body(x_ref, cnt_ref, w_ref, amat_ref, b_ref, o_ref, cs_ref):
        f = w_ref.shape[1]
        o = f // heads
        n = cnt_ref.shape[0]

        @pl.when(pl.program_id(0) == 0)
        def _():
            c = cnt_ref[...]
            cs_ref[...] = c * (1.0 / jnp.max(c))

        xb = x_ref[0]
        h = jnp.dot(xb.astype(jnp.bfloat16), w_ref[...].astype(jnp.bfloat16),
                    preferred_element_type=jnp.float32)
        hb = h.astype(jnp.bfloat16)
        ones = jnp.ones((n, 1), jnp.bfloat16)
        hext = jnp.concatenate(
            sum([[hb[:, hd * o:(hd + 1) * o], ones] for hd in range(heads)],
                []), axis=1)
        al = jnp.dot(hb, amat_ref[...], preferred_element_type=jnp.float32)
        alt = al.T
        maxs = jnp.max(al, axis=0, keepdims=True)
        cnt = cs_ref[...]
        for hd in range(heads):
            a_s = alt[hd:hd + 1, :]
            a_d = al[:, heads + hd:heads + hd + 1]
            tq = a_d + maxs[0:1, hd:hd + 1]
            q = -jnp.maximum(tq, _NEG_SLOPE * tq)
            t = a_d + a_s
            ex = jnp.exp(jnp.maximum(t, _NEG_SLOPE * t) + q) * cnt
            od = jnp.dot(ex.astype(jnp.bfloat16),
                         hext[:, hd * (o + 1):(hd + 1) * (o + 1)],
                         preferred_element_type=jnp.float32)
            den = od[:, o:o + 1]
            ob = (od[:, :o] * (1.0 / (den + 1e-16))
                  + b_ref[0, hd * o:(hd + 1) * o][None, :])
            o_ref[0, :, hd * o:(hd + 1) * o] = jnp.maximum(ob, _NEG_SLOPE * ob)

    return _gat_body


def kernel(x, edge_index, W, att_src, att_dst, bias):
    B, C, T, D = x.shape
    F = W.shape[1]
    H = att_src.shape[0]
    O = F // H
    N = C
    E = edge_index.shape[1]
    ei = edge_index.astype(jnp.int32)

    ch = min(512, E)
    pad = (-E) % ch
    if pad:
        ei = jnp.concatenate(
            [ei, jnp.full((2, pad), -1, jnp.int32)], axis=1)
    nch = (E + pad) // ch
    srcs = ei[0].reshape(nch, 1, ch)
    dsts = ei[1].reshape(nch, 1, ch)

    cnt = pl.pallas_call(
        _count_body,
        grid=(nch,),
        in_specs=[
            pl.BlockSpec((1, 1, ch), lambda i: (i, 0, 0)),
            pl.BlockSpec((1, 1, ch), lambda i: (i, 0, 0)),
        ],
        out_specs=pl.BlockSpec((N, N), lambda i: (0, 0)),
        out_shape=jax.ShapeDtypeStruct((N, N), jnp.float32),
    )(srcs, dsts)

    asf = att_src.reshape(F)
    adf = att_dst.reshape(F)
    fhead = jnp.arange(F, dtype=jnp.int32) // O
    sel = (fhead[:, None] == jnp.arange(H, dtype=jnp.int32)[None, :])
    amat = jnp.concatenate(
        [jnp.where(sel, asf[:, None], 0.0), jnp.where(sel, adf[:, None], 0.0)],
        axis=1).astype(jnp.bfloat16)
    bf = bias.reshape(1, F)

    xv = x.reshape(B, C, T * D)
    out = pl.pallas_call(
        _make_gat_body(H),
        grid=(B * T,),
        in_specs=[
            pl.BlockSpec((1, C, D), lambda i: (i // T, 0, i % T)),
            pl.BlockSpec((N, N), lambda i: (0, 0)),
            pl.BlockSpec((D, F), lambda i: (0, 0)),
            pl.BlockSpec((F, 2 * H), lambda i: (0, 0)),
            pl.BlockSpec((1, F), lambda i: (0, 0)),
        ],
        out_specs=pl.BlockSpec((1, C, F), lambda i: (i // T, 0, i % T)),
        out_shape=jax.ShapeDtypeStruct((B, C, T * F), jnp.float32),
        scratch_shapes=[pltpu.VMEM((N, N), jnp.float32)],
    )(xv, cnt, W, amat, bf)
    return out.reshape(B, C, T, F)


# speedup vs baseline: 22.0295x; 1.0417x over previous
import jax
import jax.numpy as jnp
from jax import lax
from jax.experimental import pallas as pl
from jax.experimental.pallas import tpu as pltpu
from jax.experimental.pallas import tpu_sc as plsc

_NEG_SLOPE = 0.2
_LANES = 16


def _make_sc_count(n, e_pad, rows_per_w):
    def body(src_hbm, dst_hbm, out_hbm, src_v, dst_v, tab_v):
        info = plsc.get_sparse_core_info()
        nc = info.num_cores
        wid = lax.axis_index("s") * nc + lax.axis_index("c")
        lo = wid * rows_per_w
        pltpu.sync_copy(src_hbm, src_v)
        pltpu.sync_copy(dst_hbm, dst_v)

        zeros = jnp.zeros((_LANES,), jnp.float32)

        def zero_row(r, carry):
            for cc in range(n // _LANES):
                tab_v[r, pl.ds(cc * _LANES, _LANES)] = zeros
            return carry

        lax.fori_loop(0, rows_per_w, zero_row, 0)

        lane = lax.iota(jnp.int32, 16)
        cbase, _ = plsc.scan_count(lane)
        base = jnp.max(cbase)

        plsc.addupdate_scatter(tab_v, [lane, lo + lane],
                               jnp.ones((_LANES,), jnp.float32))

        def step(j, carry):
            d = dst_v[pl.ds(j * _LANES, _LANES)]
            s = src_v[pl.ds(j * _LANES, _LANES)]
            dl = d - lo
            valid = (dl >= 0) & (dl < rows_per_w)
            key = jnp.where(valid, dl * n + s, rows_per_w * n + lane)
            c, last = plsc.scan_count(key)
            val = (c - base + 1).astype(jnp.float32)
            plsc.addupdate_scatter(
                tab_v,
                [jnp.where(valid, dl, 0), jnp.where(valid, s, 0)],
                val, mask=valid & last)
            return carry

        lax.fori_loop(0, e_pad // _LANES, step, 0)
        pltpu.sync_copy(tab_v, out_hbm.at[pl.ds(lo, rows_per_w), :])

    mesh = plsc.VectorSubcoreMesh(core_axis_name="c", subcore_axis_name="s")
    return pl.kernel(
        body, mesh=mesh,
        compiler_params=pltpu.CompilerParams(needs_layout_passes=False),
        out_type=jax.ShapeDtypeStruct((n, n), jnp.float32),
        scratch_types=[
            pltpu.VMEM((e_pad,), jnp.int32),
            pltpu.VMEM((e_pad,), jnp.int32),
            pltpu.VMEM((rows_per_w, n), jnp.float32),
        ])


def _make_gat_body(heads):
    def _gat_body(x_ref, cnt_ref, w_ref, amat_ref, b_ref, o_ref, cs_ref):
        f = w_ref.shape[1]
        o = f // heads
        n = cnt_ref.shape[0]

        @pl.when(pl.program_id(0) == 0)
        def _():
            c = cnt_ref[...]
            cs_ref[...] = c * (1.0 / jnp.max(c))

        xb = x_ref[0]
        h = jnp.dot(xb.astype(jnp.bfloat16), w_ref[...].astype(jnp.bfloat16),
                    preferred_element_type=jnp.float32)
        hb = h.astype(jnp.bfloat16)
        ones = jnp.ones((n, 1), jnp.bfloat16)
        hext = jnp.concatenate(
            sum([[hb[:, hd * o:(hd + 1) * o], ones] for hd in range(heads)],
                []), axis=1)
        al = jnp.dot(hb, amat_ref[...], preferred_element_type=jnp.float32)
        alt = al.T
        maxs = jnp.max(al, axis=0, keepdims=True)
        cnt = cs_ref[...]
        for hd in range(heads):
            a_s = alt[hd:hd + 1, :]
            a_d = al[:, heads + hd:heads + hd + 1]
            tq = a_d + maxs[0:1, hd:hd + 1]
            q = -jnp.maximum(tq, _NEG_SLOPE * tq)
            t = a_d + a_s
            ex = jnp.exp(jnp.maximum(t, _NEG_SLOPE * t) + q) * cnt
            od = jnp.dot(ex.astype(jnp.bfloat16),
                         hext[:, hd * (o + 1):(hd + 1) * (o + 1)],
                         preferred_element_type=jnp.float32)
            den = od[:, o:o + 1]
            ob = (od[:, :o] * (1.0 / (den + 1e-16))
                  + b_ref[0, hd * o:(hd + 1) * o][None, :])
            o_ref[0, :, hd * o:(hd + 1) * o] = jnp.maximum(ob, _NEG_SLOPE * ob)

    return _gat_body


def kernel(x, edge_index, W, att_src, att_dst, bias):
    B, C, T, D = x.shape
    F = W.shape[1]
    H = att_src.shape[0]
    O = F // H
    N = C
    E = edge_index.shape[1]
    ei = edge_index.astype(jnp.int32)

    info = plsc.get_sparse_core_info()
    n_workers = info.num_cores * info.num_subcores
    rows_per_w = N // n_workers
    pad = (-E) % _LANES
    if pad:
        ei = jnp.concatenate(
            [ei, jnp.full((2, pad), -1, jnp.int32)], axis=1)
    cnt = _make_sc_count(N, E + pad, rows_per_w)(ei[0], ei[1])

    asf = att_src.reshape(F)
    adf = att_dst.reshape(F)
    fhead = jnp.arange(F, dtype=jnp.int32) // O
    sel = (fhead[:, None] == jnp.arange(H, dtype=jnp.int32)[None, :])
    amat = jnp.concatenate(
        [jnp.where(sel, asf[:, None], 0.0), jnp.where(sel, adf[:, None], 0.0)],
        axis=1).astype(jnp.bfloat16)
    bf = bias.reshape(1, F)

    xv = x.reshape(B, C, T * D)
    out = pl.pallas_call(
        _make_gat_body(H),
        grid=(B * T,),
        in_specs=[
            pl.BlockSpec((1, C, D), lambda i: (i // T, 0, i % T)),
            pl.BlockSpec((N, N), lambda i: (0, 0)),
            pl.BlockSpec((D, F), lambda i: (0, 0)),
            pl.BlockSpec((F, 2 * H), lambda i: (0, 0)),
            pl.BlockSpec((1, F), lambda i: (0, 0)),
        ],
        out_specs=pl.BlockSpec((1, C, F), lambda i: (i // T, 0, i % T)),
        out_shape=jax.ShapeDtypeStruct((B, C, T * F), jnp.float32),
        scratch_shapes=[pltpu.VMEM((N, N), jnp.float32)],
    )(xv, cnt, W, amat, bf)
    return out.reshape(B, C, T, F)


# speedup vs baseline: 25.1015x; 1.1395x over previous
import jax
import jax.numpy as jnp
from jax import lax
from jax.experimental import pallas as pl
from jax.experimental.pallas import tpu as pltpu
from jax.experimental.pallas import tpu_sc as plsc

_NEG_SLOPE = 0.2
_LANES = 16


def _make_sc_count(n, e_pad, rows_per_w):
    def body(src_hbm, dst_hbm, out_hbm, src_v, dst_v, tab_v):
        info = plsc.get_sparse_core_info()
        nc = info.num_cores
        wid = lax.axis_index("s") * nc + lax.axis_index("c")
        lo = wid * rows_per_w
        pltpu.sync_copy(src_hbm, src_v)
        pltpu.sync_copy(dst_hbm, dst_v)

        zeros = jnp.zeros((_LANES,), jnp.float32)

        def zero_row(r, carry):
            for cc in range(n // _LANES):
                tab_v[r, pl.ds(cc * _LANES, _LANES)] = zeros
            return carry

        lax.fori_loop(0, rows_per_w, zero_row, 0)

        lane = lax.iota(jnp.int32, 16)
        cbase, _ = plsc.scan_count(lane)
        base = jnp.max(cbase)

        plsc.addupdate_scatter(tab_v, [lane, lo + lane],
                               jnp.ones((_LANES,), jnp.float32))

        def step(j, carry):
            d = dst_v[pl.ds(j * _LANES, _LANES)]
            s = src_v[pl.ds(j * _LANES, _LANES)]
            dl = d - lo
            valid = (dl >= 0) & (dl < rows_per_w)
            key = jnp.where(valid, dl * n + s, rows_per_w * n + lane)
            c, last = plsc.scan_count(key)
            val = (c - base + 1).astype(jnp.float32)
            plsc.addupdate_scatter(
                tab_v,
                [jnp.where(valid, dl, 0), jnp.where(valid, s, 0)],
                val, mask=valid & last)
            return carry

        lax.fori_loop(0, e_pad // _LANES, step, 0)
        pltpu.sync_copy(tab_v, out_hbm.at[pl.ds(lo, rows_per_w), :])

    mesh = plsc.VectorSubcoreMesh(core_axis_name="c", subcore_axis_name="s")
    return pl.kernel(
        body, mesh=mesh,
        compiler_params=pltpu.CompilerParams(needs_layout_passes=False),
        out_type=jax.ShapeDtypeStruct((n, n), jnp.float32),
        scratch_types=[
            pltpu.VMEM((e_pad,), jnp.int32),
            pltpu.VMEM((e_pad,), jnp.int32),
            pltpu.VMEM((rows_per_w, n), jnp.float32),
        ])


def _make_gat_body(heads):
    def _gat_body(x_ref, cnt_ref, w_ref, amat_ref, b_ref, o_ref, cs_ref):
        f = w_ref.shape[1]
        o = f // heads
        n = cnt_ref.shape[0]

        @pl.when(pl.program_id(0) == 0)
        def _():
            c = cnt_ref[...]
            cs_ref[...] = c * (1.0 / jnp.max(c))

        xb = x_ref[0]
        h = jnp.dot(xb.astype(jnp.bfloat16), w_ref[...].astype(jnp.bfloat16),
                    preferred_element_type=jnp.float32)
        hb = h.astype(jnp.bfloat16)
        ones = jnp.ones((n, 1), jnp.bfloat16)
        hext = jnp.concatenate(
            sum([[hb[:, hd * o:(hd + 1) * o], ones] for hd in range(heads)],
                []), axis=1)
        al = jnp.dot(hb, amat_ref[...], preferred_element_type=jnp.float32)
        alt = al.T
        maxs = jnp.max(al, axis=0, keepdims=True)
        cnt = cs_ref[...]
        for hd in range(heads):
            a_s = alt[hd:hd + 1, :]
            a_d = al[:, heads + hd:heads + hd + 1]
            u = a_d + maxs[0:1, hd:hd + 1]
            q = -jnp.maximum(u, _NEG_SLOPE * u)
            rp = jnp.exp(u + q)
            rn = jnp.exp(_NEG_SLOPE * u + q)
            sm = a_s - maxs[0:1, hd:hd + 1]
            cp = jnp.exp(sm)
            cn = jnp.exp(_NEG_SLOPE * sm)
            ex = jnp.maximum(rp * cp, rn * cn) * cnt
            od = jnp.dot(ex.astype(jnp.bfloat16),
                         hext[:, hd * (o + 1):(hd + 1) * (o + 1)],
                         preferred_element_type=jnp.float32)
            den = od[:, o:o + 1]
            ob = (od[:, :o] * (1.0 / (den + 1e-16))
                  + b_ref[0, hd * o:(hd + 1) * o][None, :])
            o_ref[0, :, hd * o:(hd + 1) * o] = jnp.maximum(ob, _NEG_SLOPE * ob)

    return _gat_body


def kernel(x, edge_index, W, att_src, att_dst, bias):
    B, C, T, D = x.shape
    F = W.shape[1]
    H = att_src.shape[0]
    O = F // H
    N = C
    E = edge_index.shape[1]
    ei = edge_index.astype(jnp.int32)

    info = plsc.get_sparse_core_info()
    n_workers = info.num_cores * info.num_subcores
    rows_per_w = N // n_workers
    pad = (-E) % _LANES
    if pad:
        ei = jnp.concatenate(
            [ei, jnp.full((2, pad), -1, jnp.int32)], axis=1)
    cnt = _make_sc_count(N, E + pad, rows_per_w)(ei[0], ei[1])

    asf = att_src.reshape(F)
    adf = att_dst.reshape(F)
    fhead = jnp.arange(F, dtype=jnp.int32) // O
    sel = (fhead[:, None] == jnp.arange(H, dtype=jnp.int32)[None, :])
    amat = jnp.concatenate(
        [jnp.where(sel, asf[:, None], 0.0), jnp.where(sel, adf[:, None], 0.0)],
        axis=1).astype(jnp.bfloat16)
    bf = bias.reshape(1, F)

    xv = x.reshape(B, C, T * D)
    out = pl.pallas_call(
        _make_gat_body(H),
        grid=(B * T,),
        in_specs=[
            pl.BlockSpec((1, C, D), lambda i: (i // T, 0, i % T)),
            pl.BlockSpec((N, N), lambda i: (0, 0)),
            pl.BlockSpec((D, F), lambda i: (0, 0)),
            pl.BlockSpec((F, 2 * H), lambda i: (0, 0)),
            pl.BlockSpec((1, F), lambda i: (0, 0)),
        ],
        out_specs=pl.BlockSpec((1, C, F), lambda i: (i // T, 0, i % T)),
        out_shape=jax.ShapeDtypeStruct((B, C, T * F), jnp.float32),
        scratch_shapes=[pltpu.VMEM((N, N), jnp.float32)],
    )(xv, cnt, W, amat, bf)
    return out.reshape(B, C, T, F)


# speedup vs baseline: 30.2086x; 1.2035x over previous
import jax
import jax.numpy as jnp
from jax import lax
from jax.experimental import pallas as pl
from jax.experimental.pallas import tpu as pltpu
from jax.experimental.pallas import tpu_sc as plsc

_NEG_SLOPE = 0.2
_LANES = 16


def _make_sc_count(n, e_pad, rows_per_w):
    def body(src_hbm, dst_hbm, out_hbm, src_v, dst_v, tab_v):
        info = plsc.get_sparse_core_info()
        nc = info.num_cores
        wid = lax.axis_index("s") * nc + lax.axis_index("c")
        lo = wid * rows_per_w
        pltpu.sync_copy(src_hbm, src_v)
        pltpu.sync_copy(dst_hbm, dst_v)

        zeros = jnp.zeros((_LANES,), jnp.float32)

        def zero_row(r, carry):
            for cc in range(n // _LANES):
                tab_v[r, pl.ds(cc * _LANES, _LANES)] = zeros
            return carry

        lax.fori_loop(0, rows_per_w, zero_row, 0)

        lane = lax.iota(jnp.int32, 16)
        cbase, _ = plsc.scan_count(lane)
        base = jnp.max(cbase)

        plsc.addupdate_scatter(tab_v, [lane, lo + lane],
                               jnp.ones((_LANES,), jnp.float32))

        def step(j, carry):
            d = dst_v[pl.ds(j * _LANES, _LANES)]
            s = src_v[pl.ds(j * _LANES, _LANES)]
            dl = d - lo
            valid = (dl >= 0) & (dl < rows_per_w)
            key = jnp.where(valid, dl * n + s, rows_per_w * n + lane)
            c, last = plsc.scan_count(key)
            val = (c - base + 1).astype(jnp.float32)
            plsc.addupdate_scatter(
                tab_v,
                [jnp.where(valid, dl, 0), jnp.where(valid, s, 0)],
                val, mask=valid & last)
            return carry

        lax.fori_loop(0, e_pad // _LANES, step, 0)
        pltpu.sync_copy(tab_v, out_hbm.at[pl.ds(lo, rows_per_w), :])

    mesh = plsc.VectorSubcoreMesh(core_axis_name="c", subcore_axis_name="s")
    return pl.kernel(
        body, mesh=mesh,
        compiler_params=pltpu.CompilerParams(needs_layout_passes=False),
        out_type=jax.ShapeDtypeStruct((n, n), jnp.float32),
        scratch_types=[
            pltpu.VMEM((e_pad,), jnp.int32),
            pltpu.VMEM((e_pad,), jnp.int32),
            pltpu.VMEM((rows_per_w, n), jnp.float32),
        ])


def _make_gat_body(heads):
    def _gat_body(x_ref, cnt_ref, w_ref, amat_ref, b_ref, o_ref, cs_ref):
        f = w_ref.shape[1]
        o = f // heads
        n = cnt_ref.shape[0]

        @pl.when(pl.program_id(0) == 0)
        def _():
            c = cnt_ref[...]
            cs_ref[...] = (c * (1.0 / jnp.max(c))).astype(jnp.bfloat16)

        xb = x_ref[0]
        h = jnp.dot(xb.astype(jnp.bfloat16), w_ref[...].astype(jnp.bfloat16),
                    preferred_element_type=jnp.float32)
        hb = h.astype(jnp.bfloat16)
        ones = jnp.ones((n, 1), jnp.bfloat16)
        hext = jnp.concatenate(
            sum([[hb[:, hd * o:(hd + 1) * o], ones] for hd in range(heads)],
                []), axis=1)
        al = jnp.dot(hb, amat_ref[...], preferred_element_type=jnp.float32)
        alt = al.T
        maxs = jnp.max(al, axis=0, keepdims=True)
        cnt = cs_ref[...]
        for hd in range(heads):
            a_s = alt[hd:hd + 1, :]
            a_d = al[:, heads + hd:heads + hd + 1]
            u = a_d + maxs[0:1, hd:hd + 1]
            q = -jnp.maximum(u, _NEG_SLOPE * u)
            rp = jnp.exp(u + q).astype(jnp.bfloat16)
            rn = jnp.exp(_NEG_SLOPE * u + q).astype(jnp.bfloat16)
            sm = a_s - maxs[0:1, hd:hd + 1]
            cp = jnp.exp(sm).astype(jnp.bfloat16)
            cn = jnp.exp(_NEG_SLOPE * sm).astype(jnp.bfloat16)
            ex = jnp.maximum(rp * cp, rn * cn) * cnt
            od = jnp.dot(ex,
                         hext[:, hd * (o + 1):(hd + 1) * (o + 1)],
                         preferred_element_type=jnp.float32)
            den = od[:, o:o + 1]
            ob = (od[:, :o] * (1.0 / (den + 1e-16))
                  + b_ref[0, hd * o:(hd + 1) * o][None, :])
            o_ref[0, :, hd * o:(hd + 1) * o] = jnp.maximum(ob, _NEG_SLOPE * ob)

    return _gat_body


def kernel(x, edge_index, W, att_src, att_dst, bias):
    B, C, T, D = x.shape
    F = W.shape[1]
    H = att_src.shape[0]
    O = F // H
    N = C
    E = edge_index.shape[1]
    ei = edge_index.astype(jnp.int32)

    info = plsc.get_sparse_core_info()
    n_workers = info.num_cores * info.num_subcores
    rows_per_w = N // n_workers
    pad = (-E) % _LANES
    if pad:
        ei = jnp.concatenate(
            [ei, jnp.full((2, pad), -1, jnp.int32)], axis=1)
    cnt = _make_sc_count(N, E + pad, rows_per_w)(ei[0], ei[1])

    asf = att_src.reshape(F)
    adf = att_dst.reshape(F)
    fhead = jnp.arange(F, dtype=jnp.int32) // O
    sel = (fhead[:, None] == jnp.arange(H, dtype=jnp.int32)[None, :])
    amat = jnp.concatenate(
        [jnp.where(sel, asf[:, None], 0.0), jnp.where(sel, adf[:, None], 0.0)],
        axis=1).astype(jnp.bfloat16)
    bf = bias.reshape(1, F)

    xv = x.reshape(B, C, T * D)
    out = pl.pallas_call(
        _make_gat_body(H),
        grid=(B * T,),
        in_specs=[
            pl.BlockSpec((1, C, D), lambda i: (i // T, 0, i % T)),
            pl.BlockSpec((N, N), lambda i: (0, 0)),
            pl.BlockSpec((D, F), lambda i: (0, 0)),
            pl.BlockSpec((F, 2 * H), lambda i: (0, 0)),
            pl.BlockSpec((1, F), lambda i: (0, 0)),
        ],
        out_specs=pl.BlockSpec((1, C, F), lambda i: (i // T, 0, i % T)),
        out_shape=jax.ShapeDtypeStruct((B, C, T * F), jnp.float32),
        scratch_shapes=[pltpu.VMEM((N, N), jnp.bfloat16)],
    )(xv, cnt, W, amat, bf)
    return out.reshape(B, C, T, F)
